# baseline (speedup 1.0000x reference)
import functools
import math

import jax
import jax.numpy as jnp
from jax import lax
from jax.experimental import pallas as pl
from jax.experimental.pallas import tpu as pltpu
from jax.experimental.pallas import tpu_sc as plsc

EMBED_DIM = 128
SCALE = float(math.sqrt(EMBED_DIM))

NC = 2
NS = 16
NW = NC * NS
LANES = 16

N_ROWS = 4096
N_COLS = 50
PAD_COLS = 56
ROWS_PER_TILE = N_ROWS // NW
NBUF = 8
N_OUTER = ROWS_PER_TILE // NBUF
GAHEAD = 4


def _emb_body(idx_hbm, table_hbm, out_hbm, idx_v, rows, gsem, ssem):
    c = lax.axis_index("c")
    s = lax.axis_index("s")
    wid = s * NC + c
    base = wid * ROWS_PER_TILE

    pltpu.sync_copy(idx_hbm.at[pl.ds(wid * ROWS_PER_TILE * PAD_COLS,
                                     ROWS_PER_TILE * PAD_COLS)], idx_v)

    def gather_start(r, buf):
        pltpu.async_copy(table_hbm.at[idx_v.at[pl.ds(r * PAD_COLS, PAD_COLS)]],
                         rows[buf], gsem)

    def gather_wait():
        pltpu.make_async_copy(
            table_hbm.at[idx_v.at[pl.ds(0, PAD_COLS)]], rows[0], gsem).wait()

    def scatter_start(r, buf):
        pltpu.async_copy(rows[buf], out_hbm.at[base + r], ssem)

    def scatter_wait():
        pltpu.make_async_copy(rows[0], out_hbm.at[0], ssem).wait()

    def scale(buf):
        @pl.loop(0, PAD_COLS, unroll=7)
        def _(j):
            for col in range(EMBED_DIM // LANES):
                sl = pl.ds(col * LANES, LANES)
                rows[buf][j, sl] = rows[buf][j, sl] * SCALE

    for r in range(GAHEAD):
        gather_start(r, r)

    @pl.loop(0, N_OUTER)
    def _(o):
        for b in range(NBUF):
            r = o * NBUF + b
            gather_wait()
            if b >= NBUF - GAHEAD:
                scatter_wait()
            else:
                @pl.when(o > 0)
                def _():
                    scatter_wait()
            if NBUF * (N_OUTER - 1) + b + GAHEAD < ROWS_PER_TILE:
                gather_start(r + GAHEAD, (b + GAHEAD) % NBUF)
            else:
                @pl.when(o < N_OUTER - 1)
                def _():
                    gather_start(r + GAHEAD, (b + GAHEAD) % NBUF)
            scale(b)
            scatter_start(r, b)

    for _ in range(NBUF - GAHEAD):
        scatter_wait()


@jax.jit
def _emb_call(idx, table):
    mesh = plsc.VectorSubcoreMesh(core_axis_name="c", subcore_axis_name="s",
                                  num_cores=NC, num_subcores=NS)
    fn = pl.kernel(
        _emb_body,
        out_type=jax.ShapeDtypeStruct((N_ROWS, PAD_COLS, EMBED_DIM),
                                      jnp.float32),
        mesh=mesh,
        scratch_types=[
            pltpu.VMEM((ROWS_PER_TILE * PAD_COLS,), jnp.int32),
            [pltpu.VMEM((PAD_COLS, EMBED_DIM), jnp.float32)
             for _ in range(NBUF)],
            pltpu.SemaphoreType.DMA,
            pltpu.SemaphoreType.DMA,
        ],
    )
    return fn(idx, table)[:, :N_COLS, :]


def kernel(input, table):
    idx = jnp.asarray(input, jnp.int32)
    idx = jnp.pad(idx, ((0, 0), (0, PAD_COLS - N_COLS))).reshape(-1)
    return _emb_call(idx, table)


# speedup vs baseline: 1.0002x; 1.0002x over previous
import functools
import math

import jax
import jax.numpy as jnp
from jax import lax
from jax.experimental import pallas as pl
from jax.experimental.pallas import tpu as pltpu
from jax.experimental.pallas import tpu_sc as plsc

EMBED_DIM = 128
SCALE = float(math.sqrt(EMBED_DIM))

NC = 2
NS = 16
NW = NC * NS
LANES = 16

N_ROWS = 4096
N_COLS = 50
PAD_COLS = 56
B_TOTAL = N_ROWS * PAD_COLS
B_PER_W = B_TOTAL // NW
CHUNK = 128
NCHUNK = B_PER_W // CHUNK
NBUF = 7
N_OUTER = NCHUNK // NBUF
GAHEAD = 3


def _emb_body(idx_hbm, table_hbm, out_hbm, idx_v, rows, gsem, ssem):
    c = lax.axis_index("c")
    s = lax.axis_index("s")
    wid = s * NC + c
    base = wid * B_PER_W

    pltpu.sync_copy(idx_hbm.at[wid], idx_v)

    def gather_start(g, buf):
        pltpu.async_copy(table_hbm.at[idx_v.at[g]], rows[buf], gsem)

    def gather_wait():
        pltpu.make_async_copy(table_hbm.at[idx_v.at[0]], rows[0], gsem).wait()

    def scatter_start(g, buf):
        pltpu.async_copy(rows[buf], out_hbm.at[pl.ds(base + g * CHUNK, CHUNK)],
                         ssem)

    def scatter_wait():
        pltpu.make_async_copy(rows[0], out_hbm.at[pl.ds(base, CHUNK)],
                              ssem).wait()

    def scale(buf):
        @pl.loop(0, CHUNK, unroll=8)
        def _(r):
            for col in range(EMBED_DIM // LANES):
                sl = pl.ds(col * LANES, LANES)
                rows[buf][r, sl] = rows[buf][r, sl] * SCALE

    for g in range(GAHEAD):
        gather_start(g, g)

    @pl.loop(0, N_OUTER)
    def _(o):
        for b in range(NBUF):
            g = o * NBUF + b
            gather_wait()
            if b >= NBUF - GAHEAD:
                scatter_wait()
            else:
                @pl.when(o > 0)
                def _():
                    scatter_wait()
            if NBUF * (N_OUTER - 1) + b + GAHEAD < NCHUNK:
                gather_start(g + GAHEAD, (b + GAHEAD) % NBUF)
            else:
                @pl.when(o < N_OUTER - 1)
                def _():
                    gather_start(g + GAHEAD, (b + GAHEAD) % NBUF)
            scale(b)
            scatter_start(g, b)

    for _ in range(NBUF - GAHEAD):
        scatter_wait()


@jax.jit
def _emb_call(idx, table):
    mesh = plsc.VectorSubcoreMesh(core_axis_name="c", subcore_axis_name="s",
                                  num_cores=NC, num_subcores=NS)
    fn = pl.kernel(
        _emb_body,
        out_type=jax.ShapeDtypeStruct((B_TOTAL, EMBED_DIM), jnp.float32),
        mesh=mesh,
        scratch_types=[
            pltpu.VMEM((NCHUNK, CHUNK), jnp.int32),
            [pltpu.VMEM((CHUNK, EMBED_DIM), jnp.float32) for _ in range(NBUF)],
            pltpu.SemaphoreType.DMA,
            pltpu.SemaphoreType.DMA,
        ],
    )
    out = fn(idx, table)
    return out.reshape(N_ROWS, PAD_COLS, EMBED_DIM)[:, :N_COLS, :]


def kernel(input, table):
    idx = jnp.asarray(input, jnp.int32)
    idx = jnp.pad(idx, ((0, 0), (0, PAD_COLS - N_COLS)))
    idx = idx.reshape(NW, NCHUNK, CHUNK)
    return _emb_call(idx, table)


# speedup vs baseline: 6.6121x; 6.6106x over previous
import functools
import math

import jax
import jax.numpy as jnp
from jax import lax
from jax.experimental import pallas as pl
from jax.experimental.pallas import tpu as pltpu
from jax.experimental.pallas import tpu_sc as plsc

EMBED_DIM = 128
SCALE = float(math.sqrt(EMBED_DIM))

NC = 2
NS = 16
NW = NC * NS
LANES = 16

N_ROWS = 4096
N_COLS = 50
PAD_COLS = 56
B_TOTAL = N_ROWS * PAD_COLS
B_PER_W = B_TOTAL // NW
CHUNK = 128
NCHUNK = B_PER_W // CHUNK
NBUF = 7
N_OUTER = NCHUNK // NBUF
GAHEAD = 3


def _emb_body(idx_hbm, table_hbm, out_hbm, idx_v, rows, gsem, ssem):
    c = lax.axis_index("c")
    s = lax.axis_index("s")
    wid = s * NC + c
    base = wid * B_PER_W

    pltpu.sync_copy(idx_hbm.at[wid], idx_v)

    def gather_start(g, buf):
        pltpu.async_copy(table_hbm.at[idx_v.at[g]], rows[buf], gsem)

    def gather_wait():
        pltpu.make_async_copy(table_hbm.at[idx_v.at[0]], rows[0], gsem).wait()

    def scatter_start(g, buf):
        pltpu.async_copy(rows[buf], out_hbm.at[pl.ds(base + g * CHUNK, CHUNK)],
                         ssem)

    def scatter_wait():
        pltpu.make_async_copy(rows[0], out_hbm.at[pl.ds(base, CHUNK)],
                              ssem).wait()

    def scale(buf):
        @pl.loop(0, CHUNK, unroll=8)
        def _(r):
            for col in range(EMBED_DIM // LANES):
                sl = pl.ds(col * LANES, LANES)
                rows[buf][r, sl] = rows[buf][r, sl] * SCALE

    for g in range(GAHEAD):
        gather_start(g, g)

    @pl.loop(0, N_OUTER)
    def _(o):
        for b in range(NBUF):
            g = o * NBUF + b
            gather_wait()
            if b >= NBUF - GAHEAD:
                scatter_wait()
            else:
                @pl.when(o > 0)
                def _():
                    scatter_wait()
            if NBUF * (N_OUTER - 1) + b + GAHEAD < NCHUNK:
                gather_start(g + GAHEAD, (b + GAHEAD) % NBUF)
            else:
                @pl.when(o < N_OUTER - 1)
                def _():
                    gather_start(g + GAHEAD, (b + GAHEAD) % NBUF)
            scale(b)
            scatter_start(g, b)

    for _ in range(NBUF - GAHEAD):
        scatter_wait()


@jax.jit
def _emb_call(idx, table):
    mesh = plsc.VectorSubcoreMesh(core_axis_name="c", subcore_axis_name="s",
                                  num_cores=NC, num_subcores=NS)
    fn = pl.kernel(
        _emb_body,
        out_type=jax.ShapeDtypeStruct((B_TOTAL, EMBED_DIM), jnp.float32),
        mesh=mesh,
        scratch_types=[
            pltpu.VMEM((NCHUNK, CHUNK), jnp.int32),
            [pltpu.VMEM((CHUNK, EMBED_DIM), jnp.float32) for _ in range(NBUF)],
            pltpu.SemaphoreType.DMA,
            pltpu.SemaphoreType.DMA,
        ],
    )
    out = fn(idx, table)
    return out.reshape(N_ROWS, PAD_COLS, EMBED_DIM)[:, :N_COLS, :]


def kernel(input, table):
    idx = jnp.asarray(input, jnp.int32)
    vocab = table.shape[0]
    spread = (lax.broadcasted_iota(jnp.int32, (N_ROWS, PAD_COLS - N_COLS), 0)
              * (PAD_COLS - N_COLS)
              + lax.broadcasted_iota(jnp.int32, (N_ROWS, PAD_COLS - N_COLS), 1)
              ) * 521 % vocab
    idx = jnp.concatenate([idx, spread], axis=1)
    idx = idx.reshape(NW, NCHUNK, CHUNK)
    return _emb_call(idx, table)


# speedup vs baseline: 7.6397x; 1.1554x over previous
import functools
import math

import jax
import jax.numpy as jnp
from jax import lax
from jax.experimental import pallas as pl
from jax.experimental.pallas import tpu as pltpu
from jax.experimental.pallas import tpu_sc as plsc

EMBED_DIM = 128
SCALE = float(math.sqrt(EMBED_DIM))

NC = 2
NS = 16
NW = NC * NS
LANES = 16

N_ROWS = 4096
N_COLS = 50
PAD_COLS = 56
ROWS_PER_TILE = N_ROWS // NW
RPC = 2
CLOOK = RPC * PAD_COLS
NCHUNK = ROWS_PER_TILE // RPC
NBUF = 8
N_OUTER = NCHUNK // NBUF
GAHEAD = 4


def _emb_body(idx_hbm, table_hbm, out_hbm, idx_v, rows, gsem, ssem):
    c = lax.axis_index("c")
    s = lax.axis_index("s")
    wid = s * NC + c
    base = wid * ROWS_PER_TILE

    pltpu.sync_copy(idx_hbm.at[pl.ds(wid * ROWS_PER_TILE * PAD_COLS,
                                     ROWS_PER_TILE * PAD_COLS)], idx_v)

    def gather_start(g, buf):
        pltpu.async_copy(table_hbm.at[idx_v.at[pl.ds(g * CLOOK, CLOOK)]],
                         rows[buf], gsem)

    def gather_wait():
        pltpu.make_async_copy(
            table_hbm.at[idx_v.at[pl.ds(0, CLOOK)]], rows[0], gsem).wait()

    def scatter_start(g, buf):
        for m in range(RPC):
            i = base + g * RPC + m
            pltpu.async_copy(rows[buf].at[pl.ds(PAD_COLS * m, 48)],
                             out_hbm.at[i, pl.ds(0, 48)], ssem)
            pltpu.async_copy(rows[buf].at[pl.ds(PAD_COLS * m + 48, 2)],
                             out_hbm.at[i, pl.ds(48, 2)], ssem)

    def scatter_wait():
        for _ in range(RPC):
            pltpu.make_async_copy(rows[0].at[pl.ds(0, 48)],
                                  out_hbm.at[0, pl.ds(0, 48)], ssem).wait()
            pltpu.make_async_copy(rows[0].at[pl.ds(48, 2)],
                                  out_hbm.at[0, pl.ds(48, 2)], ssem).wait()

    def scale(buf):
        @pl.loop(0, CLOOK, unroll=8)
        def _(j):
            for col in range(EMBED_DIM // LANES):
                sl = pl.ds(col * LANES, LANES)
                rows[buf][j, sl] = rows[buf][j, sl] * SCALE

    for g in range(GAHEAD):
        gather_start(g, g)

    @pl.loop(0, N_OUTER)
    def _(o):
        for b in range(NBUF):
            g = o * NBUF + b
            gather_wait()
            if b >= NBUF - GAHEAD:
                scatter_wait()
            else:
                @pl.when(o > 0)
                def _():
                    scatter_wait()
            if NBUF * (N_OUTER - 1) + b + GAHEAD < NCHUNK:
                gather_start(g + GAHEAD, (b + GAHEAD) % NBUF)
            else:
                @pl.when(o < N_OUTER - 1)
                def _():
                    gather_start(g + GAHEAD, (b + GAHEAD) % NBUF)
            scale(b)
            scatter_start(g, b)

    for _ in range(NBUF - GAHEAD):
        scatter_wait()


@jax.jit
def _emb_call(idx, table):
    mesh = plsc.VectorSubcoreMesh(core_axis_name="c", subcore_axis_name="s",
                                  num_cores=NC, num_subcores=NS)
    fn = pl.kernel(
        _emb_body,
        out_type=jax.ShapeDtypeStruct((N_ROWS, N_COLS, EMBED_DIM),
                                      jnp.float32),
        mesh=mesh,
        scratch_types=[
            pltpu.VMEM((ROWS_PER_TILE * PAD_COLS,), jnp.int32),
            [pltpu.VMEM((CLOOK, EMBED_DIM), jnp.float32)
             for _ in range(NBUF)],
            pltpu.SemaphoreType.DMA,
            pltpu.SemaphoreType.DMA,
        ],
        compiler_params=pltpu.CompilerParams(use_tc_tiling_on_sc=True),
    )
    return fn(idx, table)


def kernel(input, table):
    idx = jnp.asarray(input, jnp.int32)
    vocab = table.shape[0]
    spread = (lax.broadcasted_iota(jnp.int32, (N_ROWS, PAD_COLS - N_COLS), 0)
              * (PAD_COLS - N_COLS)
              + lax.broadcasted_iota(jnp.int32, (N_ROWS, PAD_COLS - N_COLS), 1)
              ) * 521 % vocab
    idx = jnp.concatenate([idx, spread], axis=1).reshape(-1)
    return _emb_call(idx, table)
